# baseline (device time: 34876 ns/iter reference)
import functools

import jax
import jax.numpy as jnp
from jax import lax
from jax.experimental import pallas as pl
from jax.experimental.pallas import tpu as pltpu

B, S, NH, D = 4, 512, 8, 64
K = NH * D
N = 1024
S_HALF = S // 2

_KN_DOT = (((0,), (0,)), ((), ()))


def kernel(O, Wo):
    OT = jnp.transpose(O, (0, 2, 3, 1)).reshape(B, K, S)

    def body(
        o_hbm,
        w_hbm,
        out_hbm,
        o_vmem,
        w_vmem,
        acc,
        send_buf,
        recv_buf,
        in_sems,
        out_sems,
        send_sems,
        recv_sems,
    ):
        my_x = lax.axis_index("x")
        my_y = lax.axis_index("y")
        my_z = lax.axis_index("z")
        peer = (my_x, 1 - my_y, my_z)

        o_cp = pltpu.make_async_copy(o_hbm, o_vmem, in_sems.at[0])
        w_cp = pltpu.make_async_copy(w_hbm, w_vmem, in_sems.at[1])
        o_cp.start()
        w_cp.start()

        barrier = pltpu.get_barrier_semaphore()
        pl.semaphore_signal(
            barrier, inc=1, device_id=peer, device_id_type=pl.DeviceIdType.MESH
        )
        pl.semaphore_wait(barrier, 1)
        o_cp.wait()
        w_cp.wait()

        w = w_vmem[...].astype(jnp.bfloat16)
        peer_s0 = (1 - my_y) * S_HALF
        my_s0 = my_y * S_HALF

        rdmas = []
        for b in range(B):
            a = o_vmem[b, :, pl.ds(peer_s0, S_HALF)].astype(jnp.bfloat16)
            p = lax.dot_general(a, w, _KN_DOT, preferred_element_type=jnp.float32)
            send_buf[b] = p.astype(jnp.bfloat16)
            r = pltpu.make_async_remote_copy(
                src_ref=send_buf.at[b],
                dst_ref=recv_buf.at[b],
                send_sem=send_sems.at[b],
                recv_sem=recv_sems.at[b],
                device_id=peer,
                device_id_type=pl.DeviceIdType.MESH,
            )
            r.start()
            rdmas.append(r)

        for b in range(B):
            a = o_vmem[b, :, pl.ds(my_s0, S_HALF)].astype(jnp.bfloat16)
            acc[b] = lax.dot_general(a, w, _KN_DOT, preferred_element_type=jnp.float32)

        out_cps = []
        for b in range(B):
            rdmas[b].wait_recv()
            acc[b] = acc[b] + recv_buf[b].astype(jnp.float32)
            cp = pltpu.make_async_copy(acc.at[b], out_hbm.at[b], out_sems.at[b])
            cp.start()
            out_cps.append(cp)

        for b in range(B):
            rdmas[b].wait_send()
            out_cps[b].wait()

        @functools.partial(pl.run_scoped, exit_sem=pltpu.SemaphoreType.REGULAR)
        def _(exit_sem):
            pl.semaphore_signal(
                exit_sem,
                inc=1,
                device_id=peer,
                device_id_type=pl.DeviceIdType.MESH,
            )
            pl.semaphore_wait(exit_sem, 1)

    return pl.pallas_call(
        body,
        out_shape=jax.ShapeDtypeStruct((B, S_HALF, N), jnp.float32),
        in_specs=[
            pl.BlockSpec(memory_space=pl.ANY),
            pl.BlockSpec(memory_space=pl.ANY),
        ],
        out_specs=pl.BlockSpec(memory_space=pl.ANY),
        scratch_shapes=[
            pltpu.VMEM((B, K, S), jnp.float32),
            pltpu.VMEM((K, N), jnp.float32),
            pltpu.VMEM((B, S_HALF, N), jnp.float32),
            pltpu.VMEM((B, S_HALF, N), jnp.bfloat16),
            pltpu.VMEM((B, S_HALF, N), jnp.bfloat16),
            pltpu.SemaphoreType.DMA((2,)),
            pltpu.SemaphoreType.DMA((B,)),
            pltpu.SemaphoreType.DMA((B,)),
            pltpu.SemaphoreType.DMA((B,)),
        ],
        compiler_params=pltpu.CompilerParams(collective_id=0),
    )(OT, Wo)


# device time: 32043 ns/iter; 1.0884x vs baseline; 1.0884x over previous
import functools

import jax
import jax.numpy as jnp
from jax import lax
from jax.experimental import pallas as pl
from jax.experimental.pallas import tpu as pltpu

B, S, NH, D = 4, 512, 8, 64
K = NH * D
N = 1024
S_HALF = S // 2

_KN_DOT = (((0,), (0,)), ((), ()))


def kernel(O, Wo):
    OT = jnp.transpose(O, (0, 2, 3, 1)).reshape(B, K, S)
    OT = pltpu.with_memory_space_constraint(OT, pltpu.MemorySpace.HBM)
    Wo = pltpu.with_memory_space_constraint(Wo, pltpu.MemorySpace.HBM)

    def body(
        o_hbm,
        w_hbm,
        out_hbm,
        o_vmem,
        w_vmem,
        acc,
        send_buf,
        recv_buf,
        in_sems,
        out_sems,
        send_sems,
        recv_sems,
    ):
        my_x = lax.axis_index("x")
        my_y = lax.axis_index("y")
        my_z = lax.axis_index("z")
        peer = (my_x, 1 - my_y, my_z)

        o_cp = pltpu.make_async_copy(o_hbm, o_vmem, in_sems.at[0])
        w_cp = pltpu.make_async_copy(w_hbm, w_vmem, in_sems.at[1])
        o_cp.start()
        w_cp.start()

        barrier = pltpu.get_barrier_semaphore()
        pl.semaphore_signal(
            barrier, inc=1, device_id=peer, device_id_type=pl.DeviceIdType.MESH
        )
        pl.semaphore_wait(barrier, 1)
        o_cp.wait()
        w_cp.wait()

        w = w_vmem[...].astype(jnp.bfloat16)
        peer_s0 = (1 - my_y) * S_HALF
        my_s0 = my_y * S_HALF

        rdmas = []
        for b in range(B):
            a = o_vmem[b, :, pl.ds(peer_s0, S_HALF)].astype(jnp.bfloat16)
            p = lax.dot_general(a, w, _KN_DOT, preferred_element_type=jnp.float32)
            send_buf[b] = p.astype(jnp.bfloat16)
            r = pltpu.make_async_remote_copy(
                src_ref=send_buf.at[b],
                dst_ref=recv_buf.at[b],
                send_sem=send_sems.at[b],
                recv_sem=recv_sems.at[b],
                device_id=peer,
                device_id_type=pl.DeviceIdType.MESH,
            )
            r.start()
            rdmas.append(r)

        for b in range(B):
            a = o_vmem[b, :, pl.ds(my_s0, S_HALF)].astype(jnp.bfloat16)
            acc[b] = lax.dot_general(a, w, _KN_DOT, preferred_element_type=jnp.float32)

        out_cps = []
        for b in range(B):
            rdmas[b].wait_recv()
            acc[b] = acc[b] + recv_buf[b].astype(jnp.float32)
            cp = pltpu.make_async_copy(acc.at[b], out_hbm.at[b], out_sems.at[b])
            cp.start()
            out_cps.append(cp)

        for b in range(B):
            rdmas[b].wait_send()
            out_cps[b].wait()

        @functools.partial(pl.run_scoped, exit_sem=pltpu.SemaphoreType.REGULAR)
        def _(exit_sem):
            pl.semaphore_signal(
                exit_sem,
                inc=1,
                device_id=peer,
                device_id_type=pl.DeviceIdType.MESH,
            )
            pl.semaphore_wait(exit_sem, 1)

    return pl.pallas_call(
        body,
        out_shape=jax.ShapeDtypeStruct((B, S_HALF, N), jnp.float32),
        in_specs=[
            pl.BlockSpec(memory_space=pltpu.MemorySpace.HBM),
            pl.BlockSpec(memory_space=pltpu.MemorySpace.HBM),
        ],
        out_specs=pl.BlockSpec(memory_space=pltpu.MemorySpace.HBM),
        scratch_shapes=[
            pltpu.VMEM((B, K, S), jnp.float32),
            pltpu.VMEM((K, N), jnp.float32),
            pltpu.VMEM((B, S_HALF, N), jnp.float32),
            pltpu.VMEM((B, S_HALF, N), jnp.bfloat16),
            pltpu.VMEM((B, S_HALF, N), jnp.bfloat16),
            pltpu.SemaphoreType.DMA((2,)),
            pltpu.SemaphoreType.DMA((B,)),
            pltpu.SemaphoreType.DMA((B,)),
            pltpu.SemaphoreType.DMA((B,)),
        ],
        compiler_params=pltpu.CompilerParams(collective_id=0),
    )(OT, Wo)


# device time: 27114 ns/iter; 1.2863x vs baseline; 1.1818x over previous
import functools

import jax
import jax.numpy as jnp
from jax import lax
from jax.experimental import pallas as pl
from jax.experimental.pallas import tpu as pltpu

B, S, NH, D = 4, 512, 8, 64
K = NH * D
N = 1024
S_HALF = S // 2
HC = S_HALF // 2

_KN_DOT = (((0,), (0,)), ((), ()))


def kernel(O, Wo):
    OT = jnp.transpose(O, (0, 2, 3, 1)).reshape(B, K, S)
    OT = pltpu.with_memory_space_constraint(OT, pltpu.MemorySpace.HBM)
    Wo = pltpu.with_memory_space_constraint(Wo, pltpu.MemorySpace.HBM)

    def body(
        o_hbm,
        w_hbm,
        out_hbm,
        o_vmem,
        w_vmem,
        acc,
        sq,
        pbuf,
        in_sems,
        out_sems,
        ysend,
        yrecv,
        xsend,
        xrecv,
        zsend,
        zrecv,
    ):
        my_x = lax.axis_index("x")
        my_y = lax.axis_index("y")
        my_z = lax.axis_index("z")
        ypeer = (my_x, 1 - my_y, my_z)
        xpeer = (1 - my_x, my_y, my_z)
        zpeer = (my_x, my_y, 1 - my_z)
        neighbors = [ypeer, xpeer, zpeer]

        qy = 2 * my_x + my_z
        qx = 2 * (1 - my_x) + my_z
        qza = 2 * my_x + (1 - my_z)
        qzb = 2 * (1 - my_x) + (1 - my_z)

        o_cp = pltpu.make_async_copy(o_hbm, o_vmem, in_sems.at[0])
        w_cp = pltpu.make_async_copy(w_hbm, w_vmem, in_sems.at[1])
        o_cp.start()
        w_cp.start()

        barrier = pltpu.get_barrier_semaphore()
        for nbr in neighbors:
            pl.semaphore_signal(
                barrier, inc=1, device_id=nbr, device_id_type=pl.DeviceIdType.MESH
            )
        pl.semaphore_wait(barrier, 3)
        o_cp.wait()
        w_cp.wait()

        w = w_vmem[...].astype(jnp.bfloat16)
        peer_s0 = (1 - my_y) * S_HALF
        my_s0 = my_y * S_HALF

        a = o_vmem[qy, :, pl.ds(peer_s0, S_HALF)].astype(jnp.bfloat16)
        p = lax.dot_general(a, w, _KN_DOT, preferred_element_type=jnp.float32)
        sq[...] = p.astype(jnp.bfloat16)

        y_rdmas = []
        for h in range(2):
            r = pltpu.make_async_remote_copy(
                src_ref=sq.at[pl.ds(h * HC, HC), :],
                dst_ref=pbuf.at[qy, pl.ds(h * HC, HC), :],
                send_sem=ysend.at[h],
                recv_sem=yrecv.at[h],
                device_id=ypeer,
                device_id_type=pl.DeviceIdType.MESH,
            )
            r.start()
            y_rdmas.append(r)

        for b in range(B):
            am = o_vmem[b, :, pl.ds(my_s0, S_HALF)].astype(jnp.bfloat16)
            acc[b] = lax.dot_general(am, w, _KN_DOT, preferred_element_type=jnp.float32)

        x_rdmas = []
        z_rdmas = []
        for h in range(2):
            y_rdmas[h].wait_recv()
            rx = pltpu.make_async_remote_copy(
                src_ref=pbuf.at[qy, pl.ds(h * HC, HC), :],
                dst_ref=pbuf.at[qy, pl.ds(h * HC, HC), :],
                send_sem=xsend.at[h],
                recv_sem=xrecv.at[h],
                device_id=xpeer,
                device_id_type=pl.DeviceIdType.MESH,
            )
            rx.start()
            x_rdmas.append(rx)
            rz = pltpu.make_async_remote_copy(
                src_ref=pbuf.at[qy, pl.ds(h * HC, HC), :],
                dst_ref=pbuf.at[qy, pl.ds(h * HC, HC), :],
                send_sem=zsend.at[h],
                recv_sem=zrecv.at[h],
                device_id=zpeer,
                device_id_type=pl.DeviceIdType.MESH,
            )
            rz.start()
            z_rdmas.append(rz)

        acc[qy] = acc[qy] + pbuf[qy].astype(jnp.float32)
        cp_y = pltpu.make_async_copy(acc.at[qy], out_hbm.at[qy], out_sems.at[0])
        cp_y.start()

        for h in range(2):
            x_rdmas[h].wait_recv()
            rz = pltpu.make_async_remote_copy(
                src_ref=pbuf.at[qx, pl.ds(h * HC, HC), :],
                dst_ref=pbuf.at[qx, pl.ds(h * HC, HC), :],
                send_sem=zsend.at[2 + h],
                recv_sem=zrecv.at[2 + h],
                device_id=zpeer,
                device_id_type=pl.DeviceIdType.MESH,
            )
            rz.start()
            z_rdmas.append(rz)

        acc[qx] = acc[qx] + pbuf[qx].astype(jnp.float32)
        cp_x = pltpu.make_async_copy(acc.at[qx], out_hbm.at[qx], out_sems.at[1])
        cp_x.start()

        for h in range(2):
            pltpu.make_async_copy(
                pbuf.at[qza, pl.ds(h * HC, HC), :],
                pbuf.at[qza, pl.ds(h * HC, HC), :],
                zrecv.at[h],
            ).wait()
        acc[qza] = acc[qza] + pbuf[qza].astype(jnp.float32)
        cp_za = pltpu.make_async_copy(acc.at[qza], out_hbm.at[qza], out_sems.at[2])
        cp_za.start()

        for h in range(2):
            pltpu.make_async_copy(
                pbuf.at[qzb, pl.ds(h * HC, HC), :],
                pbuf.at[qzb, pl.ds(h * HC, HC), :],
                zrecv.at[2 + h],
            ).wait()
        acc[qzb] = acc[qzb] + pbuf[qzb].astype(jnp.float32)
        cp_zb = pltpu.make_async_copy(acc.at[qzb], out_hbm.at[qzb], out_sems.at[3])
        cp_zb.start()

        for r in y_rdmas + x_rdmas + z_rdmas:
            r.wait_send()
        for cp in (cp_y, cp_x, cp_za, cp_zb):
            cp.wait()

        @functools.partial(pl.run_scoped, exit_sem=pltpu.SemaphoreType.REGULAR)
        def _(exit_sem):
            for nbr in neighbors:
                pl.semaphore_signal(
                    exit_sem,
                    inc=1,
                    device_id=nbr,
                    device_id_type=pl.DeviceIdType.MESH,
                )
            pl.semaphore_wait(exit_sem, 3)

    return pl.pallas_call(
        body,
        out_shape=jax.ShapeDtypeStruct((B, S_HALF, N), jnp.float32),
        in_specs=[
            pl.BlockSpec(memory_space=pltpu.MemorySpace.HBM),
            pl.BlockSpec(memory_space=pltpu.MemorySpace.HBM),
        ],
        out_specs=pl.BlockSpec(memory_space=pltpu.MemorySpace.HBM),
        scratch_shapes=[
            pltpu.VMEM((B, K, S), jnp.float32),
            pltpu.VMEM((K, N), jnp.float32),
            pltpu.VMEM((B, S_HALF, N), jnp.float32),
            pltpu.VMEM((S_HALF, N), jnp.bfloat16),
            pltpu.VMEM((B, S_HALF, N), jnp.bfloat16),
            pltpu.SemaphoreType.DMA((2,)),
            pltpu.SemaphoreType.DMA((4,)),
            pltpu.SemaphoreType.DMA((2,)),
            pltpu.SemaphoreType.DMA((2,)),
            pltpu.SemaphoreType.DMA((2,)),
            pltpu.SemaphoreType.DMA((2,)),
            pltpu.SemaphoreType.DMA((4,)),
            pltpu.SemaphoreType.DMA((4,)),
        ],
        compiler_params=pltpu.CompilerParams(collective_id=0),
    )(OT, Wo)


# device time: 25144 ns/iter; 1.3871x vs baseline; 1.0783x over previous
import functools

import jax
import jax.numpy as jnp
from jax import lax
from jax.experimental import pallas as pl
from jax.experimental.pallas import tpu as pltpu

B, S, NH, D = 4, 512, 8, 64
K = NH * D
N = 1024
S_HALF = S // 2
NSUB = 4
HC = S_HALF // NSUB

_KN_DOT = (((0,), (0,)), ((), ()))


def kernel(O, Wo):
    OT = jnp.transpose(O, (0, 2, 3, 1)).reshape(B, K, S)
    OT = pltpu.with_memory_space_constraint(OT, pltpu.MemorySpace.HBM)
    Wo = pltpu.with_memory_space_constraint(Wo, pltpu.MemorySpace.HBM)

    def body(
        o_hbm,
        w_hbm,
        out_hbm,
        o_vmem,
        w_vmem,
        acc,
        sq,
        pbuf,
        ostage,
        in_sems,
        out_sems,
        ysend,
        yrecv,
        xsend,
        xrecv,
        zsend,
        zrecv,
    ):
        my_x = lax.axis_index("x")
        my_y = lax.axis_index("y")
        my_z = lax.axis_index("z")
        ypeer = (my_x, 1 - my_y, my_z)
        xpeer = (1 - my_x, my_y, my_z)
        zpeer = (my_x, my_y, 1 - my_z)
        neighbors = [ypeer, xpeer, zpeer]

        qy = 2 * my_x + my_z
        qx = 2 * (1 - my_x) + my_z
        qza = 2 * my_x + (1 - my_z)
        qzb = 2 * (1 - my_x) + (1 - my_z)

        o_cp = pltpu.make_async_copy(o_hbm, o_vmem, in_sems.at[0])
        w_cp = pltpu.make_async_copy(w_hbm, w_vmem, in_sems.at[1])
        o_cp.start()
        w_cp.start()

        barrier = pltpu.get_barrier_semaphore()
        for nbr in neighbors:
            pl.semaphore_signal(
                barrier, inc=1, device_id=nbr, device_id_type=pl.DeviceIdType.MESH
            )
        pl.semaphore_wait(barrier, 3)
        o_cp.wait()
        w_cp.wait()

        w = w_vmem[...].astype(jnp.bfloat16)
        peer_s0 = (1 - my_y) * S_HALF
        my_s0 = my_y * S_HALF

        a = o_vmem[qy, :, pl.ds(peer_s0, S_HALF)].astype(jnp.bfloat16)
        p = lax.dot_general(a, w, _KN_DOT, preferred_element_type=jnp.float32)
        sq[...] = p.astype(jnp.bfloat16)

        y_rdmas = []
        for h in range(NSUB):
            r = pltpu.make_async_remote_copy(
                src_ref=sq.at[pl.ds(h * HC, HC), :],
                dst_ref=pbuf.at[qy, pl.ds(h * HC, HC), :],
                send_sem=ysend.at[h],
                recv_sem=yrecv.at[h],
                device_id=ypeer,
                device_id_type=pl.DeviceIdType.MESH,
            )
            r.start()
            y_rdmas.append(r)

        for b in range(B):
            am = o_vmem[b, :, pl.ds(my_s0, S_HALF)].astype(jnp.bfloat16)
            acc[b] = lax.dot_general(am, w, _KN_DOT, preferred_element_type=jnp.float32)

        x_rdmas = []
        z_rdmas = []
        for h in range(NSUB):
            y_rdmas[h].wait_recv()
            rx = pltpu.make_async_remote_copy(
                src_ref=pbuf.at[qy, pl.ds(h * HC, HC), :],
                dst_ref=pbuf.at[qy, pl.ds(h * HC, HC), :],
                send_sem=xsend.at[h],
                recv_sem=xrecv.at[h],
                device_id=xpeer,
                device_id_type=pl.DeviceIdType.MESH,
            )
            rx.start()
            x_rdmas.append(rx)
            rz = pltpu.make_async_remote_copy(
                src_ref=pbuf.at[qy, pl.ds(h * HC, HC), :],
                dst_ref=pbuf.at[qy, pl.ds(h * HC, HC), :],
                send_sem=zsend.at[h],
                recv_sem=zrecv.at[h],
                device_id=zpeer,
                device_id_type=pl.DeviceIdType.MESH,
            )
            rz.start()
            z_rdmas.append(rz)

        ostage[qy] = (acc[qy] + pbuf[qy].astype(jnp.float32)).astype(jnp.bfloat16)
        cp_y = pltpu.make_async_copy(ostage.at[qy], out_hbm.at[qy], out_sems.at[0])
        cp_y.start()

        for h in range(NSUB):
            x_rdmas[h].wait_recv()
            rz = pltpu.make_async_remote_copy(
                src_ref=pbuf.at[qx, pl.ds(h * HC, HC), :],
                dst_ref=pbuf.at[qx, pl.ds(h * HC, HC), :],
                send_sem=zsend.at[NSUB + h],
                recv_sem=zrecv.at[NSUB + h],
                device_id=zpeer,
                device_id_type=pl.DeviceIdType.MESH,
            )
            rz.start()
            z_rdmas.append(rz)

        ostage[qx] = (acc[qx] + pbuf[qx].astype(jnp.float32)).astype(jnp.bfloat16)
        cp_x = pltpu.make_async_copy(ostage.at[qx], out_hbm.at[qx], out_sems.at[1])
        cp_x.start()

        for h in range(NSUB):
            pltpu.make_async_copy(
                pbuf.at[qza, pl.ds(h * HC, HC), :],
                pbuf.at[qza, pl.ds(h * HC, HC), :],
                zrecv.at[h],
            ).wait()
        ostage[qza] = (acc[qza] + pbuf[qza].astype(jnp.float32)).astype(jnp.bfloat16)
        cp_za = pltpu.make_async_copy(ostage.at[qza], out_hbm.at[qza], out_sems.at[2])
        cp_za.start()

        for h in range(NSUB):
            pltpu.make_async_copy(
                pbuf.at[qzb, pl.ds(h * HC, HC), :],
                pbuf.at[qzb, pl.ds(h * HC, HC), :],
                zrecv.at[NSUB + h],
            ).wait()
        ostage[qzb] = (acc[qzb] + pbuf[qzb].astype(jnp.float32)).astype(jnp.bfloat16)
        cp_zb = pltpu.make_async_copy(ostage.at[qzb], out_hbm.at[qzb], out_sems.at[3])
        cp_zb.start()

        for r in y_rdmas + x_rdmas + z_rdmas:
            r.wait_send()
        for cp in (cp_y, cp_x, cp_za, cp_zb):
            cp.wait()

        @functools.partial(pl.run_scoped, exit_sem=pltpu.SemaphoreType.REGULAR)
        def _(exit_sem):
            for nbr in neighbors:
                pl.semaphore_signal(
                    exit_sem,
                    inc=1,
                    device_id=nbr,
                    device_id_type=pl.DeviceIdType.MESH,
                )
            pl.semaphore_wait(exit_sem, 3)

    return pl.pallas_call(
        body,
        out_shape=jax.ShapeDtypeStruct((B, S_HALF, N), jnp.bfloat16),
        in_specs=[
            pl.BlockSpec(memory_space=pltpu.MemorySpace.HBM),
            pl.BlockSpec(memory_space=pltpu.MemorySpace.HBM),
        ],
        out_specs=pl.BlockSpec(memory_space=pltpu.MemorySpace.HBM),
        scratch_shapes=[
            pltpu.VMEM((B, K, S), jnp.float32),
            pltpu.VMEM((K, N), jnp.float32),
            pltpu.VMEM((B, S_HALF, N), jnp.float32),
            pltpu.VMEM((S_HALF, N), jnp.bfloat16),
            pltpu.VMEM((B, S_HALF, N), jnp.bfloat16),
            pltpu.VMEM((B, S_HALF, N), jnp.bfloat16),
            pltpu.SemaphoreType.DMA((2,)),
            pltpu.SemaphoreType.DMA((4,)),
            pltpu.SemaphoreType.DMA((NSUB,)),
            pltpu.SemaphoreType.DMA((NSUB,)),
            pltpu.SemaphoreType.DMA((NSUB,)),
            pltpu.SemaphoreType.DMA((NSUB,)),
            pltpu.SemaphoreType.DMA((2 * NSUB,)),
            pltpu.SemaphoreType.DMA((2 * NSUB,)),
        ],
        compiler_params=pltpu.CompilerParams(collective_id=0),
    )(OT, Wo)


# device time: 23687 ns/iter; 1.4724x vs baseline; 1.0615x over previous
import functools

import jax
import jax.numpy as jnp
from jax import lax
from jax.experimental import pallas as pl
from jax.experimental.pallas import tpu as pltpu

B, S, NH, D = 4, 512, 8, 64
K = NH * D
N = 1024
S_HALF = S // 2
NSUB = 4
HC = S_HALF // NSUB

_KN_DOT = (((0,), (0,)), ((), ()))


def kernel(O, Wo):
    OT = jnp.transpose(O, (0, 2, 3, 1)).reshape(B, K, S)
    OT = pltpu.with_memory_space_constraint(OT, pltpu.MemorySpace.HBM)
    Wo = pltpu.with_memory_space_constraint(Wo, pltpu.MemorySpace.HBM)

    def body(
        o_hbm,
        w_hbm,
        out_hbm,
        o_vmem,
        w_vmem,
        acc,
        sq,
        pbuf,
        ostage,
        in_sems,
        out_sems,
        ysend,
        yrecv,
        xsend,
        xrecv,
        zsend,
        zrecv,
    ):
        my_x = lax.axis_index("x")
        my_y = lax.axis_index("y")
        my_z = lax.axis_index("z")
        ypeer = (my_x, 1 - my_y, my_z)
        xpeer = (1 - my_x, my_y, my_z)
        zpeer = (my_x, my_y, 1 - my_z)
        neighbors = [ypeer, xpeer, zpeer]

        qy = 2 * my_x + my_z
        qx = 2 * (1 - my_x) + my_z
        qza = 2 * my_x + (1 - my_z)
        qzb = 2 * (1 - my_x) + (1 - my_z)

        o_cp = pltpu.make_async_copy(o_hbm, o_vmem, in_sems.at[0])
        w_cp = pltpu.make_async_copy(w_hbm, w_vmem, in_sems.at[1])
        o_cp.start()
        w_cp.start()

        barrier = pltpu.get_barrier_semaphore()
        for nbr in neighbors:
            pl.semaphore_signal(
                barrier, inc=1, device_id=nbr, device_id_type=pl.DeviceIdType.MESH
            )
        pl.semaphore_wait(barrier, 3)
        o_cp.wait()
        w_cp.wait()

        w = w_vmem[...].astype(jnp.bfloat16)
        peer_s0 = (1 - my_y) * S_HALF
        my_s0 = my_y * S_HALF

        a = o_vmem[qy, :, pl.ds(peer_s0, S_HALF)].astype(jnp.bfloat16)
        p = lax.dot_general(a, w, _KN_DOT, preferred_element_type=jnp.float32)
        sq[...] = p.astype(jnp.bfloat16)

        y_rdmas = []
        for h in range(NSUB):
            r = pltpu.make_async_remote_copy(
                src_ref=sq.at[pl.ds(h * HC, HC), :],
                dst_ref=pbuf.at[qy, pl.ds(h * HC, HC), :],
                send_sem=ysend.at[h],
                recv_sem=yrecv.at[h],
                device_id=ypeer,
                device_id_type=pl.DeviceIdType.MESH,
            )
            r.start()
            y_rdmas.append(r)

        for b in range(B):
            am = o_vmem[b, :, pl.ds(my_s0, S_HALF)].astype(jnp.bfloat16)
            acc[b] = lax.dot_general(am, w, _KN_DOT, preferred_element_type=jnp.float32)

        x_rdmas = []
        z_rdmas = []
        for h in range(NSUB):
            y_rdmas[h].wait_recv()
            rx = pltpu.make_async_remote_copy(
                src_ref=pbuf.at[qy, pl.ds(h * HC, HC), :],
                dst_ref=pbuf.at[qy, pl.ds(h * HC, HC), :],
                send_sem=xsend.at[h],
                recv_sem=xrecv.at[h],
                device_id=xpeer,
                device_id_type=pl.DeviceIdType.MESH,
            )
            rx.start()
            x_rdmas.append(rx)
            rz = pltpu.make_async_remote_copy(
                src_ref=pbuf.at[qy, pl.ds(h * HC, HC), :],
                dst_ref=pbuf.at[qy, pl.ds(h * HC, HC), :],
                send_sem=zsend.at[h],
                recv_sem=zrecv.at[h],
                device_id=zpeer,
                device_id_type=pl.DeviceIdType.MESH,
            )
            rz.start()
            z_rdmas.append(rz)

        ostage[qy] = (acc[qy] + pbuf[qy].astype(jnp.float32)).astype(jnp.bfloat16)
        cp_y = pltpu.make_async_copy(ostage.at[qy], out_hbm.at[qy], out_sems.at[0])
        cp_y.start()

        for h in range(2):
            x_rdmas[h].wait_recv()
            rz = pltpu.make_async_remote_copy(
                src_ref=pbuf.at[qx, pl.ds(h * HC, HC), :],
                dst_ref=pbuf.at[qx, pl.ds(h * HC, HC), :],
                send_sem=zsend.at[NSUB + h],
                recv_sem=zrecv.at[NSUB + h],
                device_id=zpeer,
                device_id_type=pl.DeviceIdType.MESH,
            )
            rz.start()
            z_rdmas.append(rz)
        for h in range(2, NSUB):
            x_rdmas[h].wait_recv()

        ostage[qx] = (acc[qx] + pbuf[qx].astype(jnp.float32)).astype(jnp.bfloat16)
        cp_x = pltpu.make_async_copy(ostage.at[qx], out_hbm.at[qx], out_sems.at[1])
        cp_x.start()

        for h in range(2):
            pltpu.make_async_copy(
                pbuf.at[qza, pl.ds(h * HC, HC), :],
                pbuf.at[qza, pl.ds(h * HC, HC), :],
                zrecv.at[h],
            ).wait()
        x2_rdmas = []
        for h in range(2, NSUB):
            pltpu.make_async_copy(
                pbuf.at[qza, pl.ds(h * HC, HC), :],
                pbuf.at[qza, pl.ds(h * HC, HC), :],
                zrecv.at[h],
            ).wait()
            rx = pltpu.make_async_remote_copy(
                src_ref=pbuf.at[qza, pl.ds(h * HC, HC), :],
                dst_ref=pbuf.at[qza, pl.ds(h * HC, HC), :],
                send_sem=xsend.at[NSUB + h - 2],
                recv_sem=xrecv.at[NSUB + h - 2],
                device_id=xpeer,
                device_id_type=pl.DeviceIdType.MESH,
            )
            rx.start()
            x2_rdmas.append(rx)
        ostage[qza] = (acc[qza] + pbuf[qza].astype(jnp.float32)).astype(jnp.bfloat16)
        cp_za = pltpu.make_async_copy(ostage.at[qza], out_hbm.at[qza], out_sems.at[2])
        cp_za.start()

        for h in range(2):
            pltpu.make_async_copy(
                pbuf.at[qzb, pl.ds(h * HC, HC), :],
                pbuf.at[qzb, pl.ds(h * HC, HC), :],
                zrecv.at[NSUB + h],
            ).wait()
        for h in range(2, NSUB):
            pltpu.make_async_copy(
                pbuf.at[qzb, pl.ds(h * HC, HC), :],
                pbuf.at[qzb, pl.ds(h * HC, HC), :],
                xrecv.at[NSUB + h - 2],
            ).wait()
        ostage[qzb] = (acc[qzb] + pbuf[qzb].astype(jnp.float32)).astype(jnp.bfloat16)
        cp_zb = pltpu.make_async_copy(ostage.at[qzb], out_hbm.at[qzb], out_sems.at[3])
        cp_zb.start()

        for r in y_rdmas + x_rdmas + z_rdmas + x2_rdmas:
            r.wait_send()
        for cp in (cp_y, cp_x, cp_za, cp_zb):
            cp.wait()

        @functools.partial(pl.run_scoped, exit_sem=pltpu.SemaphoreType.REGULAR)
        def _(exit_sem):
            for nbr in neighbors:
                pl.semaphore_signal(
                    exit_sem,
                    inc=1,
                    device_id=nbr,
                    device_id_type=pl.DeviceIdType.MESH,
                )
            pl.semaphore_wait(exit_sem, 3)

    return pl.pallas_call(
        body,
        out_shape=jax.ShapeDtypeStruct((B, S_HALF, N), jnp.bfloat16),
        in_specs=[
            pl.BlockSpec(memory_space=pltpu.MemorySpace.HBM),
            pl.BlockSpec(memory_space=pltpu.MemorySpace.HBM),
        ],
        out_specs=pl.BlockSpec(memory_space=pltpu.MemorySpace.HBM),
        scratch_shapes=[
            pltpu.VMEM((B, K, S), jnp.float32),
            pltpu.VMEM((K, N), jnp.float32),
            pltpu.VMEM((B, S_HALF, N), jnp.float32),
            pltpu.VMEM((S_HALF, N), jnp.bfloat16),
            pltpu.VMEM((B, S_HALF, N), jnp.bfloat16),
            pltpu.VMEM((B, S_HALF, N), jnp.bfloat16),
            pltpu.SemaphoreType.DMA((2,)),
            pltpu.SemaphoreType.DMA((4,)),
            pltpu.SemaphoreType.DMA((NSUB,)),
            pltpu.SemaphoreType.DMA((NSUB,)),
            pltpu.SemaphoreType.DMA((NSUB + 2,)),
            pltpu.SemaphoreType.DMA((NSUB + 2,)),
            pltpu.SemaphoreType.DMA((NSUB + 2,)),
            pltpu.SemaphoreType.DMA((NSUB + 2,)),
        ],
        compiler_params=pltpu.CompilerParams(collective_id=0),
    )(OT, Wo)


# device time: 22771 ns/iter; 1.5316x vs baseline; 1.0402x over previous
import functools

import jax
import jax.numpy as jnp
from jax import lax
from jax.experimental import pallas as pl
from jax.experimental.pallas import tpu as pltpu

B, S, NH, D = 4, 512, 8, 64
K = NH * D
N = 1024
S_HALF = S // 2
NSUB = 4
HC = S_HALF // NSUB

_KN_DOT = (((0,), (0,)), ((), ()))


def kernel(O, Wo):
    OT = jnp.transpose(O, (0, 2, 3, 1)).reshape(B, K, S)
    OT = pltpu.with_memory_space_constraint(OT, pltpu.MemorySpace.HBM)
    Wo = pltpu.with_memory_space_constraint(Wo, pltpu.MemorySpace.HBM)

    def body(
        o_hbm,
        w_hbm,
        out_hbm,
        o_vmem,
        w_vmem,
        acc,
        sq,
        pbuf,
        ostage,
        in_sems,
        out_sems,
        ysend,
        yrecv,
        xsend,
        xrecv,
        zsend,
        zrecv,
    ):
        my_x = lax.axis_index("x")
        my_y = lax.axis_index("y")
        my_z = lax.axis_index("z")
        ypeer = (my_x, 1 - my_y, my_z)
        xpeer = (1 - my_x, my_y, my_z)
        zpeer = (my_x, my_y, 1 - my_z)
        neighbors = [ypeer, xpeer, zpeer]

        qy = 2 * my_x + my_z
        qx = 2 * (1 - my_x) + my_z
        qza = 2 * my_x + (1 - my_z)
        qzb = 2 * (1 - my_x) + (1 - my_z)

        oq_cp = pltpu.make_async_copy(o_hbm.at[qy], o_vmem.at[qy], in_sems.at[0])
        w_cp = pltpu.make_async_copy(w_hbm, w_vmem, in_sems.at[1])
        oq_cp.start()
        w_cp.start()
        rest_cps = []
        for k in range(1, B):
            bidx = lax.rem(qy + k, B)
            cp = pltpu.make_async_copy(
                o_hbm.at[bidx], o_vmem.at[bidx], in_sems.at[2]
            )
            cp.start()
            rest_cps.append(cp)

        barrier = pltpu.get_barrier_semaphore()
        for nbr in neighbors:
            pl.semaphore_signal(
                barrier, inc=1, device_id=nbr, device_id_type=pl.DeviceIdType.MESH
            )
        pl.semaphore_wait(barrier, 3)
        oq_cp.wait()
        w_cp.wait()

        w = w_vmem[...].astype(jnp.bfloat16)
        peer_s0 = (1 - my_y) * S_HALF
        my_s0 = my_y * S_HALF

        y_rdmas = []
        for g in range(NSUB // 2):
            a = o_vmem[qy, :, pl.ds(peer_s0 + g * 2 * HC, 2 * HC)].astype(
                jnp.bfloat16
            )
            p = lax.dot_general(a, w, _KN_DOT, preferred_element_type=jnp.float32)
            sq[pl.ds(g * 2 * HC, 2 * HC), :] = p.astype(jnp.bfloat16)
            for h in (2 * g, 2 * g + 1):
                r = pltpu.make_async_remote_copy(
                    src_ref=sq.at[pl.ds(h * HC, HC), :],
                    dst_ref=pbuf.at[qy, pl.ds(h * HC, HC), :],
                    send_sem=ysend.at[h],
                    recv_sem=yrecv.at[h],
                    device_id=ypeer,
                    device_id_type=pl.DeviceIdType.MESH,
                )
                r.start()
                y_rdmas.append(r)

        for cp in rest_cps:
            cp.wait()
        for b in range(B):
            am = o_vmem[b, :, pl.ds(my_s0, S_HALF)].astype(jnp.bfloat16)
            acc[b] = lax.dot_general(am, w, _KN_DOT, preferred_element_type=jnp.float32)

        x_rdmas = []
        z_rdmas = []
        for h in range(NSUB):
            y_rdmas[h].wait_recv()
            rx = pltpu.make_async_remote_copy(
                src_ref=pbuf.at[qy, pl.ds(h * HC, HC), :],
                dst_ref=pbuf.at[qy, pl.ds(h * HC, HC), :],
                send_sem=xsend.at[h],
                recv_sem=xrecv.at[h],
                device_id=xpeer,
                device_id_type=pl.DeviceIdType.MESH,
            )
            rx.start()
            x_rdmas.append(rx)
            rz = pltpu.make_async_remote_copy(
                src_ref=pbuf.at[qy, pl.ds(h * HC, HC), :],
                dst_ref=pbuf.at[qy, pl.ds(h * HC, HC), :],
                send_sem=zsend.at[h],
                recv_sem=zrecv.at[h],
                device_id=zpeer,
                device_id_type=pl.DeviceIdType.MESH,
            )
            rz.start()
            z_rdmas.append(rz)

        ostage[qy] = (acc[qy] + pbuf[qy].astype(jnp.float32)).astype(jnp.bfloat16)
        cp_y = pltpu.make_async_copy(ostage.at[qy], out_hbm.at[qy], out_sems.at[0])
        cp_y.start()

        for h in range(2):
            x_rdmas[h].wait_recv()
            rz = pltpu.make_async_remote_copy(
                src_ref=pbuf.at[qx, pl.ds(h * HC, HC), :],
                dst_ref=pbuf.at[qx, pl.ds(h * HC, HC), :],
                send_sem=zsend.at[NSUB + h],
                recv_sem=zrecv.at[NSUB + h],
                device_id=zpeer,
                device_id_type=pl.DeviceIdType.MESH,
            )
            rz.start()
            z_rdmas.append(rz)
        for h in range(2, NSUB):
            x_rdmas[h].wait_recv()

        ostage[qx] = (acc[qx] + pbuf[qx].astype(jnp.float32)).astype(jnp.bfloat16)
        cp_x = pltpu.make_async_copy(ostage.at[qx], out_hbm.at[qx], out_sems.at[1])
        cp_x.start()

        for h in range(2):
            pltpu.make_async_copy(
                pbuf.at[qza, pl.ds(h * HC, HC), :],
                pbuf.at[qza, pl.ds(h * HC, HC), :],
                zrecv.at[h],
            ).wait()
        x2_rdmas = []
        for h in range(2, NSUB):
            pltpu.make_async_copy(
                pbuf.at[qza, pl.ds(h * HC, HC), :],
                pbuf.at[qza, pl.ds(h * HC, HC), :],
                zrecv.at[h],
            ).wait()
            rx = pltpu.make_async_remote_copy(
                src_ref=pbuf.at[qza, pl.ds(h * HC, HC), :],
                dst_ref=pbuf.at[qza, pl.ds(h * HC, HC), :],
                send_sem=xsend.at[NSUB + h - 2],
                recv_sem=xrecv.at[NSUB + h - 2],
                device_id=xpeer,
                device_id_type=pl.DeviceIdType.MESH,
            )
            rx.start()
            x2_rdmas.append(rx)
        ostage[qza] = (acc[qza] + pbuf[qza].astype(jnp.float32)).astype(jnp.bfloat16)
        cp_za = pltpu.make_async_copy(ostage.at[qza], out_hbm.at[qza], out_sems.at[2])
        cp_za.start()

        for h in range(2):
            pltpu.make_async_copy(
                pbuf.at[qzb, pl.ds(h * HC, HC), :],
                pbuf.at[qzb, pl.ds(h * HC, HC), :],
                zrecv.at[NSUB + h],
            ).wait()
        for h in range(2, NSUB):
            pltpu.make_async_copy(
                pbuf.at[qzb, pl.ds(h * HC, HC), :],
                pbuf.at[qzb, pl.ds(h * HC, HC), :],
                xrecv.at[NSUB + h - 2],
            ).wait()
        ostage[qzb] = (acc[qzb] + pbuf[qzb].astype(jnp.float32)).astype(jnp.bfloat16)
        cp_zb = pltpu.make_async_copy(ostage.at[qzb], out_hbm.at[qzb], out_sems.at[3])
        cp_zb.start()

        for r in y_rdmas + x_rdmas + z_rdmas + x2_rdmas:
            r.wait_send()
        for cp in (cp_y, cp_x, cp_za, cp_zb):
            cp.wait()

        @functools.partial(pl.run_scoped, exit_sem=pltpu.SemaphoreType.REGULAR)
        def _(exit_sem):
            for nbr in neighbors:
                pl.semaphore_signal(
                    exit_sem,
                    inc=1,
                    device_id=nbr,
                    device_id_type=pl.DeviceIdType.MESH,
                )
            pl.semaphore_wait(exit_sem, 3)

    return pl.pallas_call(
        body,
        out_shape=jax.ShapeDtypeStruct((B, S_HALF, N), jnp.bfloat16),
        in_specs=[
            pl.BlockSpec(memory_space=pltpu.MemorySpace.HBM),
            pl.BlockSpec(memory_space=pltpu.MemorySpace.HBM),
        ],
        out_specs=pl.BlockSpec(memory_space=pltpu.MemorySpace.HBM),
        scratch_shapes=[
            pltpu.VMEM((B, K, S), jnp.float32),
            pltpu.VMEM((K, N), jnp.float32),
            pltpu.VMEM((B, S_HALF, N), jnp.float32),
            pltpu.VMEM((S_HALF, N), jnp.bfloat16),
            pltpu.VMEM((B, S_HALF, N), jnp.bfloat16),
            pltpu.VMEM((B, S_HALF, N), jnp.bfloat16),
            pltpu.SemaphoreType.DMA((3,)),
            pltpu.SemaphoreType.DMA((4,)),
            pltpu.SemaphoreType.DMA((NSUB,)),
            pltpu.SemaphoreType.DMA((NSUB,)),
            pltpu.SemaphoreType.DMA((NSUB + 2,)),
            pltpu.SemaphoreType.DMA((NSUB + 2,)),
            pltpu.SemaphoreType.DMA((NSUB + 2,)),
            pltpu.SemaphoreType.DMA((NSUB + 2,)),
        ],
        compiler_params=pltpu.CompilerParams(collective_id=0),
    )(OT, Wo)
